# Initial kernel scaffold; baseline (speedup 1.0000x reference)
import jax
import jax.numpy as jnp
from jax.experimental import pallas as pl


def kernel(features, edge_index, W1, b1, W2, b2, Wc, bc):
    raise NotImplementedError("write your pallas kernel here")


# speedup vs baseline: 3.7247x; 3.7247x over previous
import functools

import jax
import jax.numpy as jnp
from jax import lax
from jax.experimental import pallas as pl
from jax.experimental.pallas import tpu as pltpu
from jax.experimental.pallas import tpu_sc as plsc

N_NODES = 10000
D_FEAT = 128
HIDDEN = 128
OUT = 64

NC = 2
NS = 16
NW = NC * NS
C = 128
ACC_ROWS = 10240
ROWS_PER_TILE = ACC_ROWS // NS
TRASH = N_NODES


def _sc_body(K, feat_hbm, src_hbm, dst_hbm, parts_hbm, degs_hbm,
             src_v, dst_v, rows_v, ones_v, acc_sh, deg_sh, sem):
    c = lax.axis_index("c")
    s = lax.axis_index("s")
    wid = c * NS + s

    zero16 = jnp.zeros((16,), jnp.float32)
    one16 = jnp.ones((16,), jnp.float32)
    for k in range(C // 16):
        ones_v[pl.ds(k * 16, 16)] = one16

    def _zero_row(i, carry):
        for k in range(C // 16):
            rows_v[i, pl.ds(k * 16, 16)] = zero16
        return carry

    lax.fori_loop(0, C, _zero_row, 0)

    for k in range(ROWS_PER_TILE // C):
        base = s * ROWS_PER_TILE + k * C
        pltpu.sync_copy(rows_v, acc_sh.at[pl.ds(base, C)])
        pltpu.sync_copy(rows_v.at[0], deg_sh.at[pl.ds(base, C)])
    plsc.subcore_barrier()

    pltpu.sync_copy(src_hbm.at[pl.ds(wid * K, K)], src_v)
    pltpu.sync_copy(dst_hbm.at[pl.ds(wid * K, K)], dst_v)

    def _chunk(j, carry):
        pltpu.async_copy(feat_hbm.at[src_v.at[j]], rows_v, sem).wait()
        pltpu.sync_copy(rows_v, acc_sh.at[dst_v.at[j]], add=True)
        pltpu.sync_copy(ones_v, deg_sh.at[dst_v.at[j]], add=True)
        return carry

    lax.fori_loop(0, K, _chunk, 0)
    plsc.subcore_barrier()

    row0 = s * ROWS_PER_TILE
    pltpu.sync_copy(acc_sh.at[pl.ds(row0, ROWS_PER_TILE)],
                    parts_hbm.at[c, pl.ds(row0, ROWS_PER_TILE)])
    pltpu.sync_copy(deg_sh.at[pl.ds(row0, ROWS_PER_TILE)],
                    degs_hbm.at[c, pl.ds(row0, ROWS_PER_TILE)])


def _sc_aggregate(features, src2d, dst2d, K):
    f = pl.kernel(
        functools.partial(_sc_body, K),
        out_type=[
            jax.ShapeDtypeStruct((NC, ACC_ROWS, D_FEAT), jnp.float32),
            jax.ShapeDtypeStruct((NC, ACC_ROWS), jnp.float32),
        ],
        mesh=plsc.VectorSubcoreMesh(core_axis_name="c", subcore_axis_name="s"),
        scratch_types=[
            pltpu.VMEM((K, C), jnp.int32),
            pltpu.VMEM((K, C), jnp.int32),
            pltpu.VMEM((C, D_FEAT), jnp.float32),
            pltpu.VMEM((C,), jnp.float32),
            pltpu.VMEM_SHARED((ACC_ROWS, D_FEAT), jnp.float32),
            pltpu.VMEM_SHARED((ACC_ROWS,), jnp.float32),
            pltpu.SemaphoreType.DMA,
        ],
    )
    return f(features, src2d, dst2d)


BM = 2000


def _dense_body(f_ref, p_ref, d_ref, w1_ref, b1_ref, w2_ref, b2_ref,
                wc_ref, bc_ref, o_ref):
    agg = p_ref[0] + p_ref[1]
    deg = d_ref[:, 0:1] + d_ref[:, 1:2]
    mean = agg / jnp.maximum(deg, 1.0)
    w1 = w1_ref[...]
    h = jnp.dot(f_ref[...], w1[:D_FEAT], precision=lax.Precision.HIGHEST,
                preferred_element_type=jnp.float32)
    h += jnp.dot(mean, w1[D_FEAT:], precision=lax.Precision.HIGHEST,
                 preferred_element_type=jnp.float32)
    h = jnp.maximum(h + b1_ref[...], 0.0)
    h = jnp.dot(h, w2_ref[...], precision=lax.Precision.HIGHEST,
                preferred_element_type=jnp.float32)
    h = jnp.maximum(h + b2_ref[...], 0.0)
    o_ref[...] = jnp.dot(h, wc_ref[...], precision=lax.Precision.HIGHEST,
                         preferred_element_type=jnp.float32) + bc_ref[...]


def _dense(features, parts, degs_t, W1, b1, W2, b2, Wc, bc):
    return pl.pallas_call(
        _dense_body,
        grid=(N_NODES // BM,),
        in_specs=[
            pl.BlockSpec((BM, D_FEAT), lambda i: (i, 0)),
            pl.BlockSpec((NC, BM, D_FEAT), lambda i: (0, i, 0)),
            pl.BlockSpec((BM, NC), lambda i: (i, 0)),
            pl.BlockSpec((2 * D_FEAT, HIDDEN), lambda i: (0, 0)),
            pl.BlockSpec((1, HIDDEN), lambda i: (0, 0)),
            pl.BlockSpec((HIDDEN, HIDDEN), lambda i: (0, 0)),
            pl.BlockSpec((1, HIDDEN), lambda i: (0, 0)),
            pl.BlockSpec((HIDDEN, OUT), lambda i: (0, 0)),
            pl.BlockSpec((1, OUT), lambda i: (0, 0)),
        ],
        out_specs=pl.BlockSpec((BM, OUT), lambda i: (i, 0)),
        out_shape=jax.ShapeDtypeStruct((N_NODES, OUT), jnp.float32),
    )(features, parts, degs_t, W1, b1.reshape(1, HIDDEN),
      W2, b2.reshape(1, HIDDEN), Wc, bc.reshape(1, OUT))


def kernel(features, edge_index, W1, b1, W2, b2, Wc, bc):
    e = edge_index.shape[1]
    src = edge_index[0].astype(jnp.int32)
    dst = edge_index[1].astype(jnp.int32)
    k_chunks = -(-e // (NW * C))
    k_chunks = -(-k_chunks // 8) * 8
    pad = NW * k_chunks * C - e
    if pad:
        src = jnp.concatenate([src, jnp.zeros((pad,), jnp.int32)])
        dst = jnp.concatenate([dst, jnp.full((pad,), TRASH, jnp.int32)])
    src2d = src.reshape(NW * k_chunks, C)
    dst2d = dst.reshape(NW * k_chunks, C)
    parts, degs = _sc_aggregate(features, src2d, dst2d, k_chunks)
    degs_t = degs[:, :N_NODES].T
    return _dense(features, parts[:, :N_NODES], degs_t,
                  W1, b1, W2, b2, Wc, bc)


# speedup vs baseline: 4.1783x; 1.1218x over previous
import functools

import jax
import jax.numpy as jnp
from jax import lax
from jax.experimental import pallas as pl
from jax.experimental.pallas import tpu as pltpu
from jax.experimental.pallas import tpu_sc as plsc

N_NODES = 10000
D_FEAT = 128
HIDDEN = 128
OUT = 64

NC = 2
NS = 16
NW = NC * NS
C = 128
ACC_ROWS = 10112
ROWS_PER_TILE = ACC_ROWS // NS
DEG_ROWS = 10240
DEG_PER_TILE = DEG_ROWS // NS
TRASH = N_NODES


def _sc_body(K, feat_hbm, pck_hbm, parts_hbm, degs_hbm,
             pck_v, src_i, dst_i, rows_v, ones_v, acc_sh, deg_sh,
             sem_g, sem_s, sem_d):
    c = lax.axis_index("c")
    s = lax.axis_index("s")
    wid = c * NS + s

    pltpu.async_copy(pck_hbm.at[pl.ds(wid * K, K)], pck_v, sem_g)

    zero16 = jnp.zeros((16,), jnp.float32)
    one16 = jnp.ones((16,), jnp.float32)
    for k in range(C // 16):
        ones_v[pl.ds(k * 16, 16)] = one16

    def _zero_row(i, carry):
        for k in range(C // 16):
            rows_v[0, i, pl.ds(k * 16, 16)] = zero16
        return carry

    lax.fori_loop(0, C, _zero_row, 0)

    zoffs = [min(o, ROWS_PER_TILE - C) for o in range(0, ROWS_PER_TILE, C)]
    for off in zoffs:
        pltpu.async_copy(rows_v.at[0],
                         acc_sh.at[pl.ds(s * ROWS_PER_TILE + off, C)], sem_s)
    for k in range(DEG_PER_TILE // C):
        pltpu.async_copy(rows_v.at[0, 0],
                         deg_sh.at[pl.ds(s * DEG_PER_TILE + k * C, C)], sem_d)
    for off in zoffs:
        pltpu.make_async_copy(
            rows_v.at[0],
            acc_sh.at[pl.ds(s * ROWS_PER_TILE + off, C)], sem_s).wait()
    for k in range(DEG_PER_TILE // C):
        pltpu.make_async_copy(
            rows_v.at[0, 0],
            deg_sh.at[pl.ds(s * DEG_PER_TILE + k * C, C)], sem_d).wait()
    pltpu.make_async_copy(pck_hbm.at[pl.ds(wid * K, K)], pck_v, sem_g).wait()

    def _unpack(jj, b):
        for k in range(C // 16):
            w = pck_v[jj, pl.ds(k * 16, 16)]
            src_i[b, pl.ds(k * 16, 16)] = w & 0xFFFF
            dst_i[b, pl.ds(k * 16, 16)] = w >> 16

    _unpack(0, 0)
    pltpu.async_copy(feat_hbm.at[src_i.at[0]], rows_v.at[0], sem_g)
    plsc.subcore_barrier()

    def _wait_scatter(b):
        pltpu.make_async_copy(rows_v.at[b], acc_sh.at[dst_i.at[b]],
                              sem_s).wait()
        pltpu.make_async_copy(ones_v, deg_sh.at[dst_i.at[b]], sem_d).wait()

    def _chunk(j, carry):
        b = j & 1

        @pl.when(j >= 1)
        def _():
            _wait_scatter(1 - b)

        @pl.when(j + 1 < K)
        def _():
            _unpack(j + 1, 1 - b)
            pltpu.async_copy(feat_hbm.at[src_i.at[1 - b]],
                             rows_v.at[1 - b], sem_g)

        pltpu.make_async_copy(feat_hbm.at[src_i.at[b]], rows_v.at[b],
                              sem_g).wait()
        pltpu.async_copy(rows_v.at[b], acc_sh.at[dst_i.at[b]], sem_s,
                         add=True)
        pltpu.async_copy(ones_v, deg_sh.at[dst_i.at[b]], sem_d, add=True)
        return carry

    lax.fori_loop(0, K, _chunk, 0)
    _wait_scatter((K - 1) & 1)
    plsc.subcore_barrier()

    row0 = s * ROWS_PER_TILE
    pltpu.sync_copy(acc_sh.at[pl.ds(row0, ROWS_PER_TILE)],
                    parts_hbm.at[c, pl.ds(row0, ROWS_PER_TILE)])
    deg0 = s * DEG_PER_TILE
    pltpu.sync_copy(deg_sh.at[pl.ds(deg0, DEG_PER_TILE)],
                    degs_hbm.at[c, pl.ds(deg0, DEG_PER_TILE)])


def _sc_aggregate(features, pck2d, K):
    f = pl.kernel(
        functools.partial(_sc_body, K),
        out_type=[
            jax.ShapeDtypeStruct((NC, ACC_ROWS, D_FEAT), jnp.float32),
            jax.ShapeDtypeStruct((NC, DEG_ROWS), jnp.float32),
        ],
        mesh=plsc.VectorSubcoreMesh(core_axis_name="c", subcore_axis_name="s"),
        scratch_types=[
            pltpu.VMEM((K, C), jnp.int32),
            pltpu.VMEM((2, C), jnp.int32),
            pltpu.VMEM((2, C), jnp.int32),
            pltpu.VMEM((2, C, D_FEAT), jnp.float32),
            pltpu.VMEM((C,), jnp.float32),
            pltpu.VMEM_SHARED((ACC_ROWS, D_FEAT), jnp.float32),
            pltpu.VMEM_SHARED((DEG_ROWS,), jnp.float32),
            pltpu.SemaphoreType.DMA,
            pltpu.SemaphoreType.DMA,
            pltpu.SemaphoreType.DMA,
        ],
    )
    return f(features, pck2d)


BM = 2000


def _dense_body(f_ref, p_ref, d_ref, w1_ref, b1_ref, w2_ref, b2_ref,
                wc_ref, bc_ref, o_ref):
    agg = p_ref[0] + p_ref[1]
    deg = d_ref[:, 0:1] + d_ref[:, 1:2]
    mean = agg / jnp.maximum(deg, 1.0)
    w1 = w1_ref[...]
    h = jnp.dot(f_ref[...], w1[:D_FEAT], precision=lax.Precision.HIGHEST,
                preferred_element_type=jnp.float32)
    h += jnp.dot(mean, w1[D_FEAT:], precision=lax.Precision.HIGHEST,
                 preferred_element_type=jnp.float32)
    h = jnp.maximum(h + b1_ref[...], 0.0)
    h = jnp.dot(h, w2_ref[...], precision=lax.Precision.HIGHEST,
                preferred_element_type=jnp.float32)
    h = jnp.maximum(h + b2_ref[...], 0.0)
    o_ref[...] = jnp.dot(h, wc_ref[...], precision=lax.Precision.HIGHEST,
                         preferred_element_type=jnp.float32) + bc_ref[...]


def _dense(features, parts, degs_t, W1, b1, W2, b2, Wc, bc):
    return pl.pallas_call(
        _dense_body,
        grid=(N_NODES // BM,),
        in_specs=[
            pl.BlockSpec((BM, D_FEAT), lambda i: (i, 0)),
            pl.BlockSpec((NC, BM, D_FEAT), lambda i: (0, i, 0)),
            pl.BlockSpec((BM, NC), lambda i: (i, 0)),
            pl.BlockSpec((2 * D_FEAT, HIDDEN), lambda i: (0, 0)),
            pl.BlockSpec((1, HIDDEN), lambda i: (0, 0)),
            pl.BlockSpec((HIDDEN, HIDDEN), lambda i: (0, 0)),
            pl.BlockSpec((1, HIDDEN), lambda i: (0, 0)),
            pl.BlockSpec((HIDDEN, OUT), lambda i: (0, 0)),
            pl.BlockSpec((1, OUT), lambda i: (0, 0)),
        ],
        out_specs=pl.BlockSpec((BM, OUT), lambda i: (i, 0)),
        out_shape=jax.ShapeDtypeStruct((N_NODES, OUT), jnp.float32),
    )(features, parts, degs_t, W1, b1.reshape(1, HIDDEN),
      W2, b2.reshape(1, HIDDEN), Wc, bc.reshape(1, OUT))


def kernel(features, edge_index, W1, b1, W2, b2, Wc, bc):
    e = edge_index.shape[1]
    src = edge_index[0].astype(jnp.int32)
    dst = edge_index[1].astype(jnp.int32)
    k_chunks = -(-e // (NW * C))
    k_chunks = -(-k_chunks // 8) * 8
    pad = NW * k_chunks * C - e
    if pad:
        src = jnp.concatenate([src, jnp.zeros((pad,), jnp.int32)])
        dst = jnp.concatenate([dst, jnp.full((pad,), TRASH, jnp.int32)])
    packed = (dst << 16) | src
    pck2d = packed.reshape(NW * k_chunks, C)
    parts, degs = _sc_aggregate(features, pck2d, k_chunks)
    degs_t = degs[:, :N_NODES].T
    return _dense(features, parts[:, :N_NODES], degs_t,
                  W1, b1, W2, b2, Wc, bc)


# speedup vs baseline: 12.1647x; 2.9114x over previous
import functools

import jax
import jax.numpy as jnp
from jax import lax
from jax.experimental import pallas as pl
from jax.experimental.pallas import tpu as pltpu
from jax.experimental.pallas import tpu_sc as plsc

N_NODES = 10000
D_FEAT = 128
HIDDEN = 128
OUT = 64

NC = 2
NS = 16
NW = NC * NS
C = 128
ACC_ROWS = 10112
ROWS_PER_TILE = ACC_ROWS // NS
DEG_ROWS = 10240
DEG_PER_TILE = DEG_ROWS // NS
TRASH = N_NODES


def _sc_body(K, feat_hbm, pck_hbm, parts_hbm, degs_hbm,
             pck_v, src_i, dst_i, rows_v, ones_v, acc_sh, deg_sh,
             sem_g, sem_s, sem_d):
    c = lax.axis_index("c")
    s = lax.axis_index("s")
    wid = c * NS + s

    pltpu.async_copy(pck_hbm.at[pl.ds(wid * K, K)], pck_v, sem_g)

    zero16 = jnp.zeros((16,), jnp.float32)
    one16 = jnp.ones((16,), jnp.float32)
    for k in range(C // 16):
        ones_v[pl.ds(k * 16, 16)] = one16

    def _zero_row(i, carry):
        for k in range(C // 16):
            rows_v[0, i, pl.ds(k * 16, 16)] = zero16
        return carry

    lax.fori_loop(0, C, _zero_row, 0)

    zoffs = [min(o, ROWS_PER_TILE - C) for o in range(0, ROWS_PER_TILE, C)]
    for off in zoffs:
        pltpu.async_copy(rows_v.at[0],
                         acc_sh.at[pl.ds(s * ROWS_PER_TILE + off, C)], sem_s)
    for k in range(DEG_PER_TILE // C):
        pltpu.async_copy(rows_v.at[0, 0],
                         deg_sh.at[pl.ds(s * DEG_PER_TILE + k * C, C)], sem_d)
    for off in zoffs:
        pltpu.make_async_copy(
            rows_v.at[0],
            acc_sh.at[pl.ds(s * ROWS_PER_TILE + off, C)], sem_s).wait()
    for k in range(DEG_PER_TILE // C):
        pltpu.make_async_copy(
            rows_v.at[0, 0],
            deg_sh.at[pl.ds(s * DEG_PER_TILE + k * C, C)], sem_d).wait()
    pltpu.make_async_copy(pck_hbm.at[pl.ds(wid * K, K)], pck_v, sem_g).wait()

    def _unpack(jj, b):
        for k in range(C // 16):
            w = pck_v[jj, pl.ds(k * 16, 16)]
            src_i[b, pl.ds(k * 16, 16)] = w & 0xFFFF
            dst_i[b, pl.ds(k * 16, 16)] = w >> 16

    _unpack(0, 0)
    pltpu.async_copy(feat_hbm.at[src_i.at[0]], rows_v.at[0], sem_g)
    plsc.subcore_barrier()

    def _wait_scatter(b):
        pltpu.make_async_copy(rows_v.at[b], acc_sh.at[dst_i.at[b]],
                              sem_s).wait()
        pltpu.make_async_copy(ones_v, deg_sh.at[dst_i.at[b]], sem_d).wait()

    def _chunk(j, carry):
        b = j & 1

        @pl.when(j >= 1)
        def _():
            _wait_scatter(1 - b)

        @pl.when(j + 1 < K)
        def _():
            _unpack(j + 1, 1 - b)
            pltpu.async_copy(feat_hbm.at[src_i.at[1 - b]],
                             rows_v.at[1 - b], sem_g)

        pltpu.make_async_copy(feat_hbm.at[src_i.at[b]], rows_v.at[b],
                              sem_g).wait()
        pltpu.async_copy(rows_v.at[b], acc_sh.at[dst_i.at[b]], sem_s,
                         add=True)
        pltpu.async_copy(ones_v, deg_sh.at[dst_i.at[b]], sem_d, add=True)
        return carry

    lax.fori_loop(0, K, _chunk, 0)
    _wait_scatter((K - 1) & 1)
    plsc.subcore_barrier()

    row0 = s * ROWS_PER_TILE
    pltpu.sync_copy(acc_sh.at[pl.ds(row0, ROWS_PER_TILE)],
                    parts_hbm.at[c, pl.ds(row0, ROWS_PER_TILE)])
    deg0 = s * DEG_PER_TILE
    pltpu.sync_copy(deg_sh.at[pl.ds(deg0, DEG_PER_TILE)],
                    degs_hbm.at[c, pl.ds(deg0, DEG_PER_TILE)])


def _sc_aggregate(features, pck2d, K):
    f = pl.kernel(
        functools.partial(_sc_body, K),
        out_type=[
            jax.ShapeDtypeStruct((NC, ACC_ROWS, D_FEAT), jnp.float32),
            jax.ShapeDtypeStruct((NC, DEG_ROWS), jnp.float32),
        ],
        mesh=plsc.VectorSubcoreMesh(core_axis_name="c", subcore_axis_name="s"),
        scratch_types=[
            pltpu.VMEM((K, C), jnp.int32),
            pltpu.VMEM((2, C), jnp.int32),
            pltpu.VMEM((2, C), jnp.int32),
            pltpu.VMEM((2, C, D_FEAT), jnp.float32),
            pltpu.VMEM((C,), jnp.float32),
            pltpu.VMEM_SHARED((ACC_ROWS, D_FEAT), jnp.float32),
            pltpu.VMEM_SHARED((DEG_ROWS,), jnp.float32),
            pltpu.SemaphoreType.DMA,
            pltpu.SemaphoreType.DMA,
            pltpu.SemaphoreType.DMA,
        ],
    )
    return f(features, pck2d)


BM = 2000


def _dense_body(f_ref, p_ref, d_ref, w1_ref, b1_ref, w2_ref, b2_ref,
                wc_ref, bc_ref, o_ref):
    agg = p_ref[0] + p_ref[1]
    deg = d_ref[:, 0:1] + d_ref[:, 1:2]
    mean = agg / jnp.maximum(deg, 1.0)
    w1 = w1_ref[...]
    h = jnp.dot(f_ref[...], w1[:D_FEAT], precision=lax.Precision.HIGHEST,
                preferred_element_type=jnp.float32)
    h += jnp.dot(mean, w1[D_FEAT:], precision=lax.Precision.HIGHEST,
                 preferred_element_type=jnp.float32)
    h = jnp.maximum(h + b1_ref[...], 0.0)
    h = jnp.dot(h, w2_ref[...], precision=lax.Precision.HIGHEST,
                preferred_element_type=jnp.float32)
    h = jnp.maximum(h + b2_ref[...], 0.0)
    o_ref[...] = jnp.dot(h, wc_ref[...], precision=lax.Precision.HIGHEST,
                         preferred_element_type=jnp.float32) + bc_ref[...]


def _dense(features, parts, degs_t, W1, b1, W2, b2, Wc, bc):
    return pl.pallas_call(
        _dense_body,
        grid=(N_NODES // BM,),
        in_specs=[
            pl.BlockSpec((BM, D_FEAT), lambda i: (i, 0)),
            pl.BlockSpec((NC, BM, D_FEAT), lambda i: (0, i, 0)),
            pl.BlockSpec((BM, NC), lambda i: (i, 0)),
            pl.BlockSpec((2 * D_FEAT, HIDDEN), lambda i: (0, 0)),
            pl.BlockSpec((1, HIDDEN), lambda i: (0, 0)),
            pl.BlockSpec((HIDDEN, HIDDEN), lambda i: (0, 0)),
            pl.BlockSpec((1, HIDDEN), lambda i: (0, 0)),
            pl.BlockSpec((HIDDEN, OUT), lambda i: (0, 0)),
            pl.BlockSpec((1, OUT), lambda i: (0, 0)),
        ],
        out_specs=pl.BlockSpec((BM, OUT), lambda i: (i, 0)),
        out_shape=jax.ShapeDtypeStruct((N_NODES, OUT), jnp.float32),
    )(features, parts, degs_t, W1, b1.reshape(1, HIDDEN),
      W2, b2.reshape(1, HIDDEN), Wc, bc.reshape(1, OUT))


def kernel(features, edge_index, W1, b1, W2, b2, Wc, bc):
    e = edge_index.shape[1]
    src = edge_index[0].astype(jnp.int32)
    dst = edge_index[1].astype(jnp.int32)
    k_chunks = -(-e // (NW * C))
    k_chunks = -(-k_chunks // 8) * 8
    pad = NW * k_chunks * C - e
    if pad:
        r = jnp.arange(pad, dtype=jnp.int32)
        src = jnp.concatenate([src, r % N_NODES])
        dst = jnp.concatenate([dst, TRASH + r % (ACC_ROWS - N_NODES)])
    packed = (dst << 16) | src
    pck2d = packed.reshape(NW * k_chunks, C)
    parts, degs = _sc_aggregate(features, pck2d, k_chunks)
    degs_t = degs[:, :N_NODES].T
    return _dense(features, parts[:, :N_NODES], degs_t,
                  W1, b1, W2, b2, Wc, bc)


# speedup vs baseline: 13.9611x; 1.1477x over previous
import functools

import jax
import jax.numpy as jnp
from jax import lax
from jax.experimental import pallas as pl
from jax.experimental.pallas import tpu as pltpu
from jax.experimental.pallas import tpu_sc as plsc

N_NODES = 10000
D_FEAT = 128
HIDDEN = 128
OUT = 64

NC = 2
NS = 16
NW = NC * NS
C = 128
ACC_ROWS = 10112
ROWS_PER_TILE = ACC_ROWS // NS
DEG_ROWS = 10240
DEG_PER_TILE = DEG_ROWS // NS
TRASH = N_NODES


def _sc_body(K, feat_hbm, pck_hbm, parts_hbm, degs_hbm,
             pck_v, src_i, dst_i, rows_v, ones_v, acc_sh, deg_sh,
             sem_g, sem_s, sem_d):
    c = lax.axis_index("c")
    s = lax.axis_index("s")
    wid = c * NS + s

    pltpu.async_copy(pck_hbm.at[pl.ds(wid * K, K)], pck_v, sem_g)

    zero16 = jnp.zeros((16,), jnp.float32)
    one16 = jnp.ones((16,), jnp.float32)
    for k in range(C // 16):
        ones_v[pl.ds(k * 16, 16)] = one16

    def _zero_row(i, carry):
        for k in range(C // 16):
            rows_v[0, i, pl.ds(k * 16, 16)] = zero16
        return carry

    lax.fori_loop(0, C, _zero_row, 0)

    zoffs = [min(o, ROWS_PER_TILE - C) for o in range(0, ROWS_PER_TILE, C)]
    for off in zoffs:
        pltpu.async_copy(rows_v.at[0],
                         acc_sh.at[pl.ds(s * ROWS_PER_TILE + off, C)], sem_s)
    for k in range(DEG_PER_TILE // C):
        pltpu.async_copy(rows_v.at[0, 0],
                         deg_sh.at[pl.ds(s * DEG_PER_TILE + k * C, C)], sem_d)
    for off in zoffs:
        pltpu.make_async_copy(
            rows_v.at[0],
            acc_sh.at[pl.ds(s * ROWS_PER_TILE + off, C)], sem_s).wait()
    for k in range(DEG_PER_TILE // C):
        pltpu.make_async_copy(
            rows_v.at[0, 0],
            deg_sh.at[pl.ds(s * DEG_PER_TILE + k * C, C)], sem_d).wait()
    pltpu.make_async_copy(pck_hbm.at[pl.ds(wid * K, K)], pck_v, sem_g).wait()

    def _unpack(jj, b):
        for k in range(C // 16):
            w = pck_v[jj, pl.ds(k * 16, 16)]
            src_i[b, pl.ds(k * 16, 16)] = w & 0xFFFF
            dst_i[b, pl.ds(k * 16, 16)] = w >> 16

    _unpack(0, 0)
    pltpu.async_copy(feat_hbm.at[src_i.at[0]], rows_v.at[0], sem_g)
    plsc.subcore_barrier()

    def _wait_scatter(b):
        pltpu.make_async_copy(rows_v.at[b], acc_sh.at[dst_i.at[b]],
                              sem_s).wait()
        pltpu.make_async_copy(ones_v, deg_sh.at[dst_i.at[b]], sem_d).wait()

    def _chunk(j, carry):
        b = j & 1

        @pl.when(j >= 1)
        def _():
            _wait_scatter(1 - b)

        @pl.when(j + 1 < K)
        def _():
            _unpack(j + 1, 1 - b)
            pltpu.async_copy(feat_hbm.at[src_i.at[1 - b]],
                             rows_v.at[1 - b], sem_g)

        pltpu.make_async_copy(feat_hbm.at[src_i.at[b]], rows_v.at[b],
                              sem_g).wait()
        pltpu.async_copy(rows_v.at[b], acc_sh.at[dst_i.at[b]], sem_s,
                         add=True)
        pltpu.async_copy(ones_v, deg_sh.at[dst_i.at[b]], sem_d, add=True)
        return carry

    lax.fori_loop(0, K, _chunk, 0)
    _wait_scatter((K - 1) & 1)
    plsc.subcore_barrier()

    row0 = s * ROWS_PER_TILE
    pltpu.sync_copy(acc_sh.at[pl.ds(row0, ROWS_PER_TILE)],
                    parts_hbm.at[c, pl.ds(row0, ROWS_PER_TILE)])
    deg0 = s * DEG_PER_TILE
    pltpu.sync_copy(deg_sh.at[pl.ds(deg0, DEG_PER_TILE)],
                    degs_hbm.at[c, pl.ds(deg0, DEG_PER_TILE)])


def _sc_aggregate(features, pck2d, K):
    f = pl.kernel(
        functools.partial(_sc_body, K),
        out_type=[
            jax.ShapeDtypeStruct((NC, ACC_ROWS, D_FEAT), jnp.float32),
            jax.ShapeDtypeStruct((NC, DEG_ROWS), jnp.float32),
        ],
        mesh=plsc.VectorSubcoreMesh(core_axis_name="c", subcore_axis_name="s"),
        scratch_types=[
            pltpu.VMEM((K, C), jnp.int32),
            pltpu.VMEM((2, C), jnp.int32),
            pltpu.VMEM((2, C), jnp.int32),
            pltpu.VMEM((2, C, D_FEAT), jnp.float32),
            pltpu.VMEM((C,), jnp.float32),
            pltpu.VMEM_SHARED((ACC_ROWS, D_FEAT), jnp.float32),
            pltpu.VMEM_SHARED((DEG_ROWS,), jnp.float32),
            pltpu.SemaphoreType.DMA,
            pltpu.SemaphoreType.DMA,
            pltpu.SemaphoreType.DMA,
        ],
    )
    return f(features, pck2d)


CP = 320


def _pack_body(e, src_ref, dst_ref, o_ref):
    i = pl.program_id(0)
    rows = jax.lax.broadcasted_iota(jnp.int32, (CP, 128), 0) + i * CP
    lanes = jax.lax.broadcasted_iota(jnp.int32, (CP, 128), 1)
    gid = rows * 128 + lanes
    real = (dst_ref[...] << 16) | src_ref[...]
    fake = ((TRASH + gid % (ACC_ROWS - N_NODES)) << 16) | (gid % N_NODES)
    o_ref[...] = jnp.where(gid < e, real, fake)


def _pack(src2d, dst2d, rows_out):
    grid = (rows_out // CP,)
    return pl.pallas_call(
        functools.partial(_pack_body, src2d.shape[0] * 128),
        grid=grid,
        in_specs=[
            pl.BlockSpec((CP, 128), lambda i: (i, 0)),
            pl.BlockSpec((CP, 128), lambda i: (i, 0)),
        ],
        out_specs=pl.BlockSpec((CP, 128), lambda i: (i, 0)),
        out_shape=jax.ShapeDtypeStruct((rows_out, 128), jnp.int32),
    )(src2d, dst2d)


BM = 2000


def _dense_body(f_ref, p_ref, d_ref, w1_ref, b1_ref, w2_ref, b2_ref,
                wc_ref, bc_ref, o_ref):
    agg = p_ref[0] + p_ref[1]
    deg = d_ref[:, 0:1] + d_ref[:, 1:2]
    mean = agg / jnp.maximum(deg, 1.0)
    w1 = w1_ref[...]
    h = jnp.dot(f_ref[...], w1[:D_FEAT], preferred_element_type=jnp.float32)
    h += jnp.dot(mean, w1[D_FEAT:], preferred_element_type=jnp.float32)
    h = jnp.maximum(h + b1_ref[...], 0.0)
    h = jnp.dot(h, w2_ref[...], preferred_element_type=jnp.float32)
    h = jnp.maximum(h + b2_ref[...], 0.0)
    o_ref[...] = jnp.dot(h, wc_ref[...],
                         preferred_element_type=jnp.float32) + bc_ref[...]


def _dense(features, parts, degs_t, W1, b1, W2, b2, Wc, bc):
    return pl.pallas_call(
        _dense_body,
        grid=(N_NODES // BM,),
        in_specs=[
            pl.BlockSpec((BM, D_FEAT), lambda i: (i, 0)),
            pl.BlockSpec((NC, BM, D_FEAT), lambda i: (0, i, 0)),
            pl.BlockSpec((BM, NC), lambda i: (i, 0)),
            pl.BlockSpec((2 * D_FEAT, HIDDEN), lambda i: (0, 0)),
            pl.BlockSpec((1, HIDDEN), lambda i: (0, 0)),
            pl.BlockSpec((HIDDEN, HIDDEN), lambda i: (0, 0)),
            pl.BlockSpec((1, HIDDEN), lambda i: (0, 0)),
            pl.BlockSpec((HIDDEN, OUT), lambda i: (0, 0)),
            pl.BlockSpec((1, OUT), lambda i: (0, 0)),
        ],
        out_specs=pl.BlockSpec((BM, OUT), lambda i: (i, 0)),
        out_shape=jax.ShapeDtypeStruct((N_NODES, OUT), jnp.float32),
    )(features, parts, degs_t, W1, b1.reshape(1, HIDDEN),
      W2, b2.reshape(1, HIDDEN), Wc, bc.reshape(1, OUT))


def kernel(features, edge_index, W1, b1, W2, b2, Wc, bc):
    e = edge_index.shape[1]
    assert e % C == 0
    src2d = edge_index[0].astype(jnp.int32).reshape(e // C, C)
    dst2d = edge_index[1].astype(jnp.int32).reshape(e // C, C)
    k_chunks = -(-e // (NW * C))
    k_chunks = -(-k_chunks // 8) * 8
    pck2d = _pack(src2d, dst2d, NW * k_chunks)
    parts, degs = _sc_aggregate(features, pck2d, k_chunks)
    degs_t = degs[:, :N_NODES].T
    return _dense(features, parts, degs_t, W1, b1, W2, b2, Wc, bc)


# speedup vs baseline: 15.2082x; 1.0893x over previous
import functools

import jax
import jax.numpy as jnp
from jax import lax
from jax.experimental import pallas as pl
from jax.experimental.pallas import tpu as pltpu
from jax.experimental.pallas import tpu_sc as plsc

N_NODES = 10000
D_FEAT = 128
HIDDEN = 128
OUT = 64

NC = 2
NS = 16
NW = NC * NS
C = 128
ACC_ROWS = 10112
ROWS_PER_TILE = ACC_ROWS // NS
DEG_ROWS = 10240
DEG_PER_TILE = DEG_ROWS // NS
TRASH = N_NODES


def _sc_body(K, feat_hbm, pck_hbm, parts_hbm, degs_hbm,
             pck_v, src_i, dst_i, rows_v, ones_v, acc_sh, deg_sh,
             sem_g, sem_s, sem_d):
    c = lax.axis_index("c")
    s = lax.axis_index("s")
    wid = c * NS + s

    pltpu.async_copy(pck_hbm.at[pl.ds(wid * K * C, K * C)], pck_v, sem_g)

    zero16 = jnp.zeros((16,), jnp.float32)
    one16 = jnp.ones((16,), jnp.float32)
    for k in range(C // 16):
        ones_v[pl.ds(k * 16, 16)] = one16

    def _zero_row(i, carry):
        for k in range(C // 16):
            rows_v[0, i, pl.ds(k * 16, 16)] = zero16
        return carry

    lax.fori_loop(0, C, _zero_row, 0)

    zoffs = [min(o, ROWS_PER_TILE - C) for o in range(0, ROWS_PER_TILE, C)]
    for off in zoffs:
        pltpu.async_copy(rows_v.at[0],
                         acc_sh.at[pl.ds(s * ROWS_PER_TILE + off, C)], sem_s)
    for k in range(DEG_PER_TILE // C):
        pltpu.async_copy(rows_v.at[0, 0],
                         deg_sh.at[pl.ds(s * DEG_PER_TILE + k * C, C)], sem_d)
    for off in zoffs:
        pltpu.make_async_copy(
            rows_v.at[0],
            acc_sh.at[pl.ds(s * ROWS_PER_TILE + off, C)], sem_s).wait()
    for k in range(DEG_PER_TILE // C):
        pltpu.make_async_copy(
            rows_v.at[0, 0],
            deg_sh.at[pl.ds(s * DEG_PER_TILE + k * C, C)], sem_d).wait()
    pltpu.make_async_copy(pck_hbm.at[pl.ds(wid * K * C, K * C)], pck_v,
                          sem_g).wait()

    def _unpack(jj, b):
        for k in range(C // 16):
            w = pck_v[pl.ds(jj * C + k * 16, 16)]
            src_i[b, pl.ds(k * 16, 16)] = w & 0xFFFF
            dst_i[b, pl.ds(k * 16, 16)] = w >> 16

    _unpack(0, 0)
    pltpu.async_copy(feat_hbm.at[src_i.at[0]], rows_v.at[0], sem_g)
    plsc.subcore_barrier()

    def _wait_scatter(b):
        pltpu.make_async_copy(rows_v.at[b], acc_sh.at[dst_i.at[b]],
                              sem_s).wait()
        pltpu.make_async_copy(ones_v, deg_sh.at[dst_i.at[b]], sem_d).wait()

    def _chunk(j, carry):
        b = j & 1

        @pl.when(j >= 1)
        def _():
            _wait_scatter(1 - b)

        @pl.when(j + 1 < K)
        def _():
            _unpack(j + 1, 1 - b)
            pltpu.async_copy(feat_hbm.at[src_i.at[1 - b]],
                             rows_v.at[1 - b], sem_g)

        pltpu.make_async_copy(feat_hbm.at[src_i.at[b]], rows_v.at[b],
                              sem_g).wait()
        pltpu.async_copy(rows_v.at[b], acc_sh.at[dst_i.at[b]], sem_s,
                         add=True)
        pltpu.async_copy(ones_v, deg_sh.at[dst_i.at[b]], sem_d, add=True)
        return carry

    lax.fori_loop(0, K, _chunk, 0)
    _wait_scatter((K - 1) & 1)
    plsc.subcore_barrier()

    row0 = s * ROWS_PER_TILE
    pltpu.sync_copy(acc_sh.at[pl.ds(row0, ROWS_PER_TILE)],
                    parts_hbm.at[c, pl.ds(row0, ROWS_PER_TILE)])
    deg0 = s * DEG_PER_TILE
    pltpu.sync_copy(deg_sh.at[pl.ds(deg0, DEG_PER_TILE)],
                    degs_hbm.at[c, pl.ds(deg0, DEG_PER_TILE)])


def _sc_aggregate(features, pck2d, K):
    f = pl.kernel(
        functools.partial(_sc_body, K),
        out_type=[
            jax.ShapeDtypeStruct((NC, ACC_ROWS, D_FEAT), jnp.float32),
            jax.ShapeDtypeStruct((NC, DEG_ROWS), jnp.float32),
        ],
        mesh=plsc.VectorSubcoreMesh(core_axis_name="c", subcore_axis_name="s"),
        scratch_types=[
            pltpu.VMEM((K * C,), jnp.int32),
            pltpu.VMEM((2, C), jnp.int32),
            pltpu.VMEM((2, C), jnp.int32),
            pltpu.VMEM((2, C, D_FEAT), jnp.float32),
            pltpu.VMEM((C,), jnp.float32),
            pltpu.VMEM_SHARED((ACC_ROWS, D_FEAT), jnp.float32),
            pltpu.VMEM_SHARED((DEG_ROWS,), jnp.float32),
            pltpu.SemaphoreType.DMA,
            pltpu.SemaphoreType.DMA,
            pltpu.SemaphoreType.DMA,
        ],
    )
    return f(features, pck2d)


BLK = 40960


def _pack_body(e, ei_ref, o_ref):
    i = pl.program_id(0)
    gid = jax.lax.broadcasted_iota(jnp.int32, (BLK,), 0) + i * BLK
    real = (ei_ref[1] << 16) | ei_ref[0]
    fake = ((TRASH + gid % (ACC_ROWS - N_NODES)) << 16) | (gid % N_NODES)
    o_ref[...] = jnp.where(gid < e, real, fake)


def _pack(ei, e_pad):
    return pl.pallas_call(
        functools.partial(_pack_body, ei.shape[1]),
        grid=(e_pad // BLK,),
        in_specs=[pl.BlockSpec((2, BLK), lambda i: (0, i))],
        out_specs=pl.BlockSpec((BLK,), lambda i: (i,)),
        out_shape=jax.ShapeDtypeStruct((e_pad,), jnp.int32),
    )(ei)


BM = 2560


def _dense_body(f_ref, p_ref, d_ref, w1_ref, b1_ref, w2_ref, b2_ref,
                wc_ref, bc_ref, o_ref):
    agg = p_ref[0] + p_ref[1]
    deg = jnp.reshape(d_ref[0] + d_ref[1], (BM, 1))
    mean = agg / jnp.maximum(deg, 1.0)
    w1 = w1_ref[...]
    h = jnp.dot(f_ref[...], w1[:D_FEAT], preferred_element_type=jnp.float32)
    h += jnp.dot(mean, w1[D_FEAT:], preferred_element_type=jnp.float32)
    h = jnp.maximum(h + b1_ref[...], 0.0)
    h = jnp.dot(h, w2_ref[...], preferred_element_type=jnp.float32)
    h = jnp.maximum(h + b2_ref[...], 0.0)
    o_ref[...] = jnp.dot(h, wc_ref[...],
                         preferred_element_type=jnp.float32) + bc_ref[...]


def _dense(features, parts, degs, W1, b1, W2, b2, Wc, bc):
    return pl.pallas_call(
        _dense_body,
        grid=(-(-N_NODES // BM),),
        in_specs=[
            pl.BlockSpec((BM, D_FEAT), lambda i: (i, 0)),
            pl.BlockSpec((NC, BM, D_FEAT), lambda i: (0, i, 0)),
            pl.BlockSpec((NC, BM), lambda i: (0, i)),
            pl.BlockSpec((2 * D_FEAT, HIDDEN), lambda i: (0, 0)),
            pl.BlockSpec((1, HIDDEN), lambda i: (0, 0)),
            pl.BlockSpec((HIDDEN, HIDDEN), lambda i: (0, 0)),
            pl.BlockSpec((1, HIDDEN), lambda i: (0, 0)),
            pl.BlockSpec((HIDDEN, OUT), lambda i: (0, 0)),
            pl.BlockSpec((1, OUT), lambda i: (0, 0)),
        ],
        out_specs=pl.BlockSpec((BM, OUT), lambda i: (i, 0)),
        out_shape=jax.ShapeDtypeStruct((N_NODES, OUT), jnp.float32),
    )(features, parts, degs, W1, b1.reshape(1, HIDDEN),
      W2, b2.reshape(1, HIDDEN), Wc, bc.reshape(1, OUT))


def kernel(features, edge_index, W1, b1, W2, b2, Wc, bc):
    e = edge_index.shape[1]
    k_chunks = -(-e // (NW * C))
    k_chunks = -(-k_chunks // 8) * 8
    e_pad = NW * k_chunks * C
    assert e_pad % BLK == 0
    pck = _pack(edge_index.astype(jnp.int32), e_pad)
    parts, degs = _sc_aggregate(features, pck, k_chunks)
    return _dense(features, parts, degs, W1, b1, W2, b2, Wc, bc)


# speedup vs baseline: 15.9542x; 1.0490x over previous
import functools

import jax
import jax.numpy as jnp
from jax import lax
from jax.experimental import pallas as pl
from jax.experimental.pallas import tpu as pltpu
from jax.experimental.pallas import tpu_sc as plsc

N_NODES = 10000
D_FEAT = 128
HIDDEN = 128
OUT = 64

NC = 2
NS = 16
NW = NC * NS
C = 128
ACC_ROWS = 10112
ROWS_PER_TILE = ACC_ROWS // NS
DEG_ROWS = 10240
DEG_PER_TILE = DEG_ROWS // NS
TRASH = N_NODES


def _sc_body(K, feat_hbm, pck_hbm, parts_hbm, degs_hbm,
             pck_v, src_i, dst_i, rows_v, ones_v, acc_sh, deg_sh,
             sem_g, sem_s, sem_d):
    c = lax.axis_index("c")
    s = lax.axis_index("s")
    wid = c * NS + s

    pltpu.async_copy(pck_hbm.at[pl.ds(wid * K * C, K * C)], pck_v, sem_g)

    zero16 = jnp.zeros((16,), jnp.float32)
    one16 = jnp.ones((16,), jnp.float32)
    for k in range(C // 16):
        ones_v[pl.ds(k * 16, 16)] = one16

    def _zero_row(i, carry):
        for k in range(C // 16):
            rows_v[0, i, pl.ds(k * 16, 16)] = zero16
        return carry

    lax.fori_loop(0, C, _zero_row, 0)

    zoffs = [min(o, ROWS_PER_TILE - C) for o in range(0, ROWS_PER_TILE, C)]
    for off in zoffs:
        pltpu.async_copy(rows_v.at[0],
                         acc_sh.at[pl.ds(s * ROWS_PER_TILE + off, C)], sem_s)
    for k in range(DEG_PER_TILE // C):
        pltpu.async_copy(rows_v.at[0, 0],
                         deg_sh.at[pl.ds(s * DEG_PER_TILE + k * C, C)], sem_d)
    for off in zoffs:
        pltpu.make_async_copy(
            rows_v.at[0],
            acc_sh.at[pl.ds(s * ROWS_PER_TILE + off, C)], sem_s).wait()
    for k in range(DEG_PER_TILE // C):
        pltpu.make_async_copy(
            rows_v.at[0, 0],
            deg_sh.at[pl.ds(s * DEG_PER_TILE + k * C, C)], sem_d).wait()
    pltpu.make_async_copy(pck_hbm.at[pl.ds(wid * K * C, K * C)], pck_v,
                          sem_g).wait()

    def _unpack(jj, b):
        for k in range(C // 16):
            w = pck_v[pl.ds(jj * C + k * 16, 16)]
            src_i[b, pl.ds(k * 16, 16)] = w & 0xFFFF
            dst_i[b, pl.ds(k * 16, 16)] = w >> 16

    _unpack(0, 0)
    pltpu.async_copy(feat_hbm.at[src_i.at[0]], rows_v.at[0], sem_g)
    plsc.subcore_barrier()

    def _wait_scatter(b):
        pltpu.make_async_copy(rows_v.at[b], acc_sh.at[dst_i.at[b]],
                              sem_s).wait()
        pltpu.make_async_copy(ones_v, deg_sh.at[dst_i.at[b]], sem_d).wait()

    def _chunk(j, carry):
        b = j & 1

        @pl.when(j >= 1)
        def _():
            _wait_scatter(1 - b)

        @pl.when(j + 1 < K)
        def _():
            _unpack(j + 1, 1 - b)
            pltpu.async_copy(feat_hbm.at[src_i.at[1 - b]],
                             rows_v.at[1 - b], sem_g)

        pltpu.make_async_copy(feat_hbm.at[src_i.at[b]], rows_v.at[b],
                              sem_g).wait()
        pltpu.async_copy(rows_v.at[b], acc_sh.at[dst_i.at[b]], sem_s,
                         add=True)
        pltpu.async_copy(ones_v, deg_sh.at[dst_i.at[b]], sem_d, add=True)
        return carry

    lax.fori_loop(0, K, _chunk, 0)
    _wait_scatter((K - 1) & 1)
    plsc.subcore_barrier()

    row0 = s * ROWS_PER_TILE
    pltpu.sync_copy(acc_sh.at[pl.ds(row0, ROWS_PER_TILE)],
                    parts_hbm.at[c, pl.ds(row0, ROWS_PER_TILE)])
    deg0 = s * DEG_PER_TILE
    pltpu.sync_copy(deg_sh.at[pl.ds(deg0, DEG_PER_TILE)],
                    degs_hbm.at[c, pl.ds(deg0, DEG_PER_TILE)])


def _sc_aggregate(features, pck2d, K):
    f = pl.kernel(
        functools.partial(_sc_body, K),
        out_type=[
            jax.ShapeDtypeStruct((NC, ACC_ROWS, D_FEAT), jnp.float32),
            jax.ShapeDtypeStruct((NC, DEG_ROWS), jnp.float32),
        ],
        mesh=plsc.VectorSubcoreMesh(core_axis_name="c", subcore_axis_name="s"),
        scratch_types=[
            pltpu.VMEM((K * C,), jnp.int32),
            pltpu.VMEM((2, C), jnp.int32),
            pltpu.VMEM((2, C), jnp.int32),
            pltpu.VMEM((2, C, D_FEAT), jnp.float32),
            pltpu.VMEM((C,), jnp.float32),
            pltpu.VMEM_SHARED((ACC_ROWS, D_FEAT), jnp.float32),
            pltpu.VMEM_SHARED((DEG_ROWS,), jnp.float32),
            pltpu.SemaphoreType.DMA,
            pltpu.SemaphoreType.DMA,
            pltpu.SemaphoreType.DMA,
        ],
    )
    return f(features, pck2d)


BLK = 40960


def _pack_body(e, ei_ref, o_ref):
    i = pl.program_id(0)
    gid = jax.lax.broadcasted_iota(jnp.int32, (BLK,), 0) + i * BLK
    real = (ei_ref[1] << 16) | ei_ref[0]
    fake = ((TRASH + (gid & 63)) << 16) | (gid & 8191)
    o_ref[...] = jnp.where(gid < e, real, fake)


def _pack(ei, e_pad):
    return pl.pallas_call(
        functools.partial(_pack_body, ei.shape[1]),
        grid=(e_pad // BLK,),
        in_specs=[pl.BlockSpec((2, BLK), lambda i: (0, i))],
        out_specs=pl.BlockSpec((BLK,), lambda i: (i,)),
        out_shape=jax.ShapeDtypeStruct((e_pad,), jnp.int32),
    )(ei)


BM = 2560


def _dense_body(f_ref, p_ref, d_ref, w1_ref, b1_ref, w2_ref, b2_ref,
                wc_ref, bc_ref, o_ref):
    agg = p_ref[0] + p_ref[1]
    deg = jnp.reshape(d_ref[0] + d_ref[1], (BM, 1))
    mean = agg / jnp.maximum(deg, 1.0)
    w1 = w1_ref[...]
    h = jnp.dot(f_ref[...], w1[:D_FEAT], preferred_element_type=jnp.float32)
    h += jnp.dot(mean, w1[D_FEAT:], preferred_element_type=jnp.float32)
    h = jnp.maximum(h + b1_ref[...], 0.0)
    h = jnp.dot(h, w2_ref[...], preferred_element_type=jnp.float32)
    h = jnp.maximum(h + b2_ref[...], 0.0)
    o_ref[...] = jnp.dot(h, wc_ref[...],
                         preferred_element_type=jnp.float32) + bc_ref[...]


def _dense(features, parts, degs, W1, b1, W2, b2, Wc, bc):
    return pl.pallas_call(
        _dense_body,
        grid=(-(-N_NODES // BM),),
        in_specs=[
            pl.BlockSpec((BM, D_FEAT), lambda i: (i, 0)),
            pl.BlockSpec((NC, BM, D_FEAT), lambda i: (0, i, 0)),
            pl.BlockSpec((NC, BM), lambda i: (0, i)),
            pl.BlockSpec((2 * D_FEAT, HIDDEN), lambda i: (0, 0)),
            pl.BlockSpec((1, HIDDEN), lambda i: (0, 0)),
            pl.BlockSpec((HIDDEN, HIDDEN), lambda i: (0, 0)),
            pl.BlockSpec((1, HIDDEN), lambda i: (0, 0)),
            pl.BlockSpec((HIDDEN, OUT), lambda i: (0, 0)),
            pl.BlockSpec((1, OUT), lambda i: (0, 0)),
        ],
        out_specs=pl.BlockSpec((BM, OUT), lambda i: (i, 0)),
        out_shape=jax.ShapeDtypeStruct((N_NODES, OUT), jnp.float32),
    )(features, parts, degs, W1, b1.reshape(1, HIDDEN),
      W2, b2.reshape(1, HIDDEN), Wc, bc.reshape(1, OUT))


def kernel(features, edge_index, W1, b1, W2, b2, Wc, bc):
    e = edge_index.shape[1]
    k_chunks = -(-e // (NW * C))
    k_chunks = -(-k_chunks // 8) * 8
    e_pad = NW * k_chunks * C
    assert e_pad % BLK == 0
    pck = _pack(edge_index.astype(jnp.int32), e_pad)
    parts, degs = _sc_aggregate(features, pck, k_chunks)
    return _dense(features, parts, degs, W1, b1, W2, b2, Wc, bc)


# speedup vs baseline: 15.9611x; 1.0004x over previous
import functools

import jax
import jax.numpy as jnp
from jax import lax
from jax.experimental import pallas as pl
from jax.experimental.pallas import tpu as pltpu
from jax.experimental.pallas import tpu_sc as plsc

N_NODES = 10000
D_FEAT = 128
HIDDEN = 128
OUT = 64

NC = 2
NS = 16
NW = NC * NS
C = 128
ACC_ROWS = 10112
ROWS_PER_TILE = ACC_ROWS // NS
DEG_ROWS = 10240
DEG_PER_TILE = DEG_ROWS // NS
TRASH = N_NODES


def _sc_body(K, feat_hbm, pck_hbm, parts_hbm, degs_hbm,
             pck_v, src_i, dst_i, rows_v, ones_v, acc_sh, deg_sh,
             sem_g, sem_s, sem_d):
    c = lax.axis_index("c")
    s = lax.axis_index("s")
    wid = c * NS + s

    pltpu.async_copy(pck_hbm.at[pl.ds(wid * K * C, K * C)], pck_v, sem_g)

    zero16 = jnp.zeros((16,), jnp.float32)
    one16 = jnp.ones((16,), jnp.float32)
    for k in range(C // 16):
        ones_v[pl.ds(k * 16, 16)] = one16

    def _zero_row(i, carry):
        for k in range(C // 16):
            rows_v[0, i, pl.ds(k * 16, 16)] = zero16
        return carry

    lax.fori_loop(0, C, _zero_row, 0)

    zoffs = [min(o, ROWS_PER_TILE - C) for o in range(0, ROWS_PER_TILE, C)]
    for off in zoffs:
        pltpu.async_copy(rows_v.at[0],
                         acc_sh.at[pl.ds(s * ROWS_PER_TILE + off, C)], sem_s)
    for k in range(DEG_PER_TILE // C):
        pltpu.async_copy(rows_v.at[0, 0],
                         deg_sh.at[pl.ds(s * DEG_PER_TILE + k * C, C)], sem_d)
    for off in zoffs:
        pltpu.make_async_copy(
            rows_v.at[0],
            acc_sh.at[pl.ds(s * ROWS_PER_TILE + off, C)], sem_s).wait()
    for k in range(DEG_PER_TILE // C):
        pltpu.make_async_copy(
            rows_v.at[0, 0],
            deg_sh.at[pl.ds(s * DEG_PER_TILE + k * C, C)], sem_d).wait()
    pltpu.make_async_copy(pck_hbm.at[pl.ds(wid * K * C, K * C)], pck_v,
                          sem_g).wait()

    def _unpack(jj, b):
        for k in range(C // 16):
            w = pck_v[pl.ds(jj * C + k * 16, 16)]
            src_i[b, pl.ds(k * 16, 16)] = w & 0xFFFF
            dst_i[b, pl.ds(k * 16, 16)] = w >> 16

    _unpack(0, 0)
    pltpu.async_copy(feat_hbm.at[src_i.at[0]], rows_v.at[0], sem_g)
    plsc.subcore_barrier()

    def _wait_scatter(b):
        pltpu.make_async_copy(rows_v.at[b], acc_sh.at[dst_i.at[b]],
                              sem_s).wait()
        pltpu.make_async_copy(ones_v, deg_sh.at[dst_i.at[b]], sem_d).wait()

    def _chunk(j, carry):
        b = j & 1

        @pl.when(j >= 1)
        def _():
            _wait_scatter(1 - b)

        @pl.when(j + 1 < K)
        def _():
            _unpack(j + 1, 1 - b)
            pltpu.async_copy(feat_hbm.at[src_i.at[1 - b]],
                             rows_v.at[1 - b], sem_g)

        pltpu.make_async_copy(feat_hbm.at[src_i.at[b]], rows_v.at[b],
                              sem_g).wait()
        pltpu.async_copy(rows_v.at[b], acc_sh.at[dst_i.at[b]], sem_s,
                         add=True)
        pltpu.async_copy(ones_v, deg_sh.at[dst_i.at[b]], sem_d, add=True)
        return carry

    lax.fori_loop(0, K, _chunk, 0)
    _wait_scatter((K - 1) & 1)
    plsc.subcore_barrier()

    row0 = s * ROWS_PER_TILE
    pltpu.sync_copy(acc_sh.at[pl.ds(row0, ROWS_PER_TILE)],
                    parts_hbm.at[c, pl.ds(row0, ROWS_PER_TILE)])
    deg0 = s * DEG_PER_TILE
    pltpu.sync_copy(deg_sh.at[pl.ds(deg0, DEG_PER_TILE)],
                    degs_hbm.at[c, pl.ds(deg0, DEG_PER_TILE)])


def _sc_aggregate(features, pck2d, K):
    f = pl.kernel(
        functools.partial(_sc_body, K),
        out_type=[
            jax.ShapeDtypeStruct((NC, ACC_ROWS, D_FEAT), jnp.float32),
            jax.ShapeDtypeStruct((NC, DEG_ROWS), jnp.float32),
        ],
        mesh=plsc.VectorSubcoreMesh(core_axis_name="c", subcore_axis_name="s"),
        scratch_types=[
            pltpu.VMEM((K * C,), jnp.int32),
            pltpu.VMEM((2, C), jnp.int32),
            pltpu.VMEM((2, C), jnp.int32),
            pltpu.VMEM((2, C, D_FEAT), jnp.float32),
            pltpu.VMEM((C,), jnp.float32),
            pltpu.VMEM_SHARED((ACC_ROWS, D_FEAT), jnp.float32),
            pltpu.VMEM_SHARED((DEG_ROWS,), jnp.float32),
            pltpu.SemaphoreType.DMA,
            pltpu.SemaphoreType.DMA,
            pltpu.SemaphoreType.DMA,
        ],
    )
    return f(features, pck2d)


BLK = 40960


def _pack_body(e, ei_ref, o_ref):
    i = pl.program_id(0)
    gid = jax.lax.broadcasted_iota(jnp.int32, (BLK,), 0) + i * BLK
    real = (ei_ref[1] << 16) | ei_ref[0]
    fake = ((TRASH + (gid & 63)) << 16) | (gid & 8191)
    o_ref[...] = jnp.where(gid < e, real, fake)


def _pack(ei, e_pad):
    return pl.pallas_call(
        functools.partial(_pack_body, ei.shape[1]),
        grid=(e_pad // BLK,),
        in_specs=[pl.BlockSpec((2, BLK), lambda i: (0, i))],
        out_specs=pl.BlockSpec((BLK,), lambda i: (i,)),
        out_shape=jax.ShapeDtypeStruct((e_pad,), jnp.int32),
    )(ei)


BM = 2560


def _pre_body(f_ref, w1_ref, b1_ref, t_ref):
    t_ref[...] = jnp.dot(f_ref[...], w1_ref[:D_FEAT],
                         preferred_element_type=jnp.float32) + b1_ref[...]


def _pre(features, W1, b1):
    return pl.pallas_call(
        _pre_body,
        grid=(-(-N_NODES // BM),),
        in_specs=[
            pl.BlockSpec((BM, D_FEAT), lambda i: (i, 0)),
            pl.BlockSpec((2 * D_FEAT, HIDDEN), lambda i: (0, 0)),
            pl.BlockSpec((1, HIDDEN), lambda i: (0, 0)),
        ],
        out_specs=pl.BlockSpec((BM, HIDDEN), lambda i: (i, 0)),
        out_shape=jax.ShapeDtypeStruct((N_NODES, HIDDEN), jnp.float32),
    )(features, W1, b1.reshape(1, HIDDEN))


def _dense_body(t_ref, p_ref, d_ref, w1_ref, w2_ref, b2_ref,
                wc_ref, bc_ref, o_ref):
    agg = p_ref[0] + p_ref[1]
    deg = jnp.reshape(d_ref[0] + d_ref[1], (BM, 1))
    mean = agg / jnp.maximum(deg, 1.0)
    h = t_ref[...] + jnp.dot(mean, w1_ref[D_FEAT:],
                             preferred_element_type=jnp.float32)
    h = jnp.maximum(h, 0.0)
    h = jnp.dot(h, w2_ref[...], preferred_element_type=jnp.float32)
    h = jnp.maximum(h + b2_ref[...], 0.0)
    o_ref[...] = jnp.dot(h, wc_ref[...],
                         preferred_element_type=jnp.float32) + bc_ref[...]


def _dense(t, parts, degs, W1, W2, b2, Wc, bc):
    return pl.pallas_call(
        _dense_body,
        grid=(-(-N_NODES // BM),),
        in_specs=[
            pl.BlockSpec((BM, HIDDEN), lambda i: (i, 0)),
            pl.BlockSpec((NC, BM, D_FEAT), lambda i: (0, i, 0)),
            pl.BlockSpec((NC, BM), lambda i: (0, i)),
            pl.BlockSpec((2 * D_FEAT, HIDDEN), lambda i: (0, 0)),
            pl.BlockSpec((HIDDEN, HIDDEN), lambda i: (0, 0)),
            pl.BlockSpec((1, HIDDEN), lambda i: (0, 0)),
            pl.BlockSpec((HIDDEN, OUT), lambda i: (0, 0)),
            pl.BlockSpec((1, OUT), lambda i: (0, 0)),
        ],
        out_specs=pl.BlockSpec((BM, OUT), lambda i: (i, 0)),
        out_shape=jax.ShapeDtypeStruct((N_NODES, OUT), jnp.float32),
    )(t, parts, degs, W1, W2, b2.reshape(1, HIDDEN),
      Wc, bc.reshape(1, OUT))


def kernel(features, edge_index, W1, b1, W2, b2, Wc, bc):
    e = edge_index.shape[1]
    k_chunks = -(-e // (NW * C))
    k_chunks = -(-k_chunks // 8) * 8
    e_pad = NW * k_chunks * C
    assert e_pad % BLK == 0
    pck = _pack(edge_index.astype(jnp.int32), e_pad)
    t = _pre(features, W1, b1)
    parts, degs = _sc_aggregate(features, pck, k_chunks)
    return _dense(t, parts, degs, W1, W2, b2, Wc, bc)


# speedup vs baseline: 16.6722x; 1.0446x over previous
import functools

import jax
import jax.numpy as jnp
from jax import lax
from jax.experimental import pallas as pl
from jax.experimental.pallas import tpu as pltpu
from jax.experimental.pallas import tpu_sc as plsc

N_NODES = 10000
D_FEAT = 128
HIDDEN = 128
OUT = 64

NC = 2
NS = 16
NW = NC * NS
C = 128
ACC_ROWS = 10112
ROWS_PER_TILE = ACC_ROWS // NS
DEG_ROWS = 10240
DEG_PER_TILE = DEG_ROWS // NS
TRASH = N_NODES


def _sc_body(K, feat_hbm, pck_hbm, parts_hbm, degs_hbm,
             pck_v, src_i, dst_i, rows_v, ones_v, acc_sh, deg_sh,
             sem_g, sem_s, sem_d):
    c = lax.axis_index("c")
    s = lax.axis_index("s")
    wid = c * NS + s

    pltpu.async_copy(pck_hbm.at[pl.ds(wid * K * C, K * C)], pck_v, sem_g)

    zero16 = jnp.zeros((16,), jnp.float32)
    one16 = jnp.ones((16,), jnp.float32)
    for k in range(C // 16):
        ones_v[pl.ds(k * 16, 16)] = one16

    def _zero_row(i, carry):
        for k in range(C // 16):
            rows_v[0, i, pl.ds(k * 16, 16)] = zero16
        return carry

    lax.fori_loop(0, C, _zero_row, 0)

    zoffs = [min(o, ROWS_PER_TILE - C) for o in range(0, ROWS_PER_TILE, C)]
    for off in zoffs:
        pltpu.async_copy(rows_v.at[0],
                         acc_sh.at[pl.ds(s * ROWS_PER_TILE + off, C)], sem_s)
    for k in range(DEG_PER_TILE // C):
        pltpu.async_copy(rows_v.at[0, 0],
                         deg_sh.at[pl.ds(s * DEG_PER_TILE + k * C, C)], sem_d)
    for off in zoffs:
        pltpu.make_async_copy(
            rows_v.at[0],
            acc_sh.at[pl.ds(s * ROWS_PER_TILE + off, C)], sem_s).wait()
    for k in range(DEG_PER_TILE // C):
        pltpu.make_async_copy(
            rows_v.at[0, 0],
            deg_sh.at[pl.ds(s * DEG_PER_TILE + k * C, C)], sem_d).wait()
    pltpu.make_async_copy(pck_hbm.at[pl.ds(wid * K * C, K * C)], pck_v,
                          sem_g).wait()

    def _unpack(jj, b):
        for k in range(C // 16):
            w = pck_v[pl.ds(jj * C + k * 16, 16)]
            src_i[b, pl.ds(k * 16, 16)] = w & 0xFFFF
            dst_i[b, pl.ds(k * 16, 16)] = w >> 16

    _unpack(0, 0)
    pltpu.async_copy(feat_hbm.at[src_i.at[0]], rows_v.at[0], sem_g)
    plsc.subcore_barrier()

    def _wait_scatter(b):
        pltpu.make_async_copy(rows_v.at[b], acc_sh.at[dst_i.at[b]],
                              sem_s).wait()
        pltpu.make_async_copy(ones_v, deg_sh.at[dst_i.at[b]], sem_d).wait()

    def _chunk(j, carry):
        b = j & 1

        @pl.when(j >= 1)
        def _():
            _wait_scatter(1 - b)

        @pl.when(j + 1 < K)
        def _():
            _unpack(j + 1, 1 - b)
            pltpu.async_copy(feat_hbm.at[src_i.at[1 - b]],
                             rows_v.at[1 - b], sem_g)

        pltpu.make_async_copy(feat_hbm.at[src_i.at[b]], rows_v.at[b],
                              sem_g).wait()
        pltpu.async_copy(rows_v.at[b], acc_sh.at[dst_i.at[b]], sem_s,
                         add=True)
        pltpu.async_copy(ones_v, deg_sh.at[dst_i.at[b]], sem_d, add=True)
        return carry

    lax.fori_loop(0, K, _chunk, 0)
    _wait_scatter((K - 1) & 1)
    plsc.subcore_barrier()

    row0 = s * ROWS_PER_TILE
    pltpu.sync_copy(acc_sh.at[pl.ds(row0, ROWS_PER_TILE)],
                    parts_hbm.at[c, pl.ds(row0, ROWS_PER_TILE)])
    deg0 = s * DEG_PER_TILE
    pltpu.sync_copy(deg_sh.at[pl.ds(deg0, DEG_PER_TILE)],
                    degs_hbm.at[c, pl.ds(deg0, DEG_PER_TILE)])


def _sc_aggregate(features, pck2d, K):
    f = pl.kernel(
        functools.partial(_sc_body, K),
        out_type=[
            jax.ShapeDtypeStruct((NC, ACC_ROWS, D_FEAT), jnp.float32),
            jax.ShapeDtypeStruct((NC, DEG_ROWS), jnp.float32),
        ],
        mesh=plsc.VectorSubcoreMesh(core_axis_name="c", subcore_axis_name="s"),
        scratch_types=[
            pltpu.VMEM((K * C,), jnp.int32),
            pltpu.VMEM((2, C), jnp.int32),
            pltpu.VMEM((2, C), jnp.int32),
            pltpu.VMEM((2, C, D_FEAT), jnp.float32),
            pltpu.VMEM((C,), jnp.float32),
            pltpu.VMEM_SHARED((ACC_ROWS, D_FEAT), jnp.float32),
            pltpu.VMEM_SHARED((DEG_ROWS,), jnp.float32),
            pltpu.SemaphoreType.DMA,
            pltpu.SemaphoreType.DMA,
            pltpu.SemaphoreType.DMA,
        ],
    )
    return f(features, pck2d)


BLK = 40960


def _pack_body(e, ei_ref, o_ref):
    i = pl.program_id(0)
    gid = jax.lax.broadcasted_iota(jnp.int32, (BLK,), 0) + i * BLK
    real = (ei_ref[1] << 16) | ei_ref[0]
    fake = ((TRASH + (gid & 63)) << 16) | (gid & 8191)
    o_ref[...] = jnp.where(gid < e, real, fake)


def _pack(ei, e_pad):
    return pl.pallas_call(
        functools.partial(_pack_body, ei.shape[1]),
        grid=(e_pad // BLK,),
        in_specs=[pl.BlockSpec((2, BLK), lambda i: (0, i))],
        out_specs=pl.BlockSpec((BLK,), lambda i: (i,)),
        out_shape=jax.ShapeDtypeStruct((e_pad,), jnp.int32),
    )(ei)


BM = 2560


def _pre_body(f_ref, w1_ref, b1_ref, t_ref):
    t_ref[...] = jnp.dot(f_ref[...], w1_ref[:D_FEAT],
                         preferred_element_type=jnp.float32) + b1_ref[...]


def _pre(features, W1, b1):
    return pl.pallas_call(
        _pre_body,
        grid=(-(-N_NODES // BM),),
        in_specs=[
            pl.BlockSpec((BM, D_FEAT), lambda i: (i, 0)),
            pl.BlockSpec((2 * D_FEAT, HIDDEN), lambda i: (0, 0)),
            pl.BlockSpec((1, HIDDEN), lambda i: (0, 0)),
        ],
        out_specs=pl.BlockSpec((BM, HIDDEN), lambda i: (i, 0)),
        out_shape=jax.ShapeDtypeStruct((N_NODES, HIDDEN), jnp.float32),
    )(features, W1, b1.reshape(1, HIDDEN))


def _dense_body(t_ref, p_ref, d_ref, w1_ref, w2_ref, b2_ref,
                wc_ref, bc_ref, o_ref):
    agg = p_ref[0] + p_ref[1]
    deg = jnp.reshape(d_ref[0] + d_ref[1], (BM, 1))
    mean = agg / jnp.maximum(deg, 1.0)
    h = t_ref[...] + jnp.dot(mean, w1_ref[D_FEAT:],
                             preferred_element_type=jnp.float32)
    h = jnp.maximum(h, 0.0)
    h = jnp.dot(h, w2_ref[...], preferred_element_type=jnp.float32)
    h = jnp.maximum(h + b2_ref[...], 0.0)
    res = jnp.dot(h, wc_ref[...],
                  preferred_element_type=jnp.float32) + bc_ref[...]
    o_ref[...] = res.T


def _dense(t, parts, degs, W1, W2, b2, Wc, bc):
    return pl.pallas_call(
        _dense_body,
        grid=(-(-N_NODES // BM),),
        in_specs=[
            pl.BlockSpec((BM, HIDDEN), lambda i: (i, 0)),
            pl.BlockSpec((NC, BM, D_FEAT), lambda i: (0, i, 0)),
            pl.BlockSpec((NC, BM), lambda i: (0, i)),
            pl.BlockSpec((2 * D_FEAT, HIDDEN), lambda i: (0, 0)),
            pl.BlockSpec((HIDDEN, HIDDEN), lambda i: (0, 0)),
            pl.BlockSpec((1, HIDDEN), lambda i: (0, 0)),
            pl.BlockSpec((HIDDEN, OUT), lambda i: (0, 0)),
            pl.BlockSpec((1, OUT), lambda i: (0, 0)),
        ],
        out_specs=pl.BlockSpec((OUT, BM), lambda i: (0, i)),
        out_shape=jax.ShapeDtypeStruct((OUT, N_NODES), jnp.float32),
    )(t, parts, degs, W1, W2, b2.reshape(1, HIDDEN),
      Wc, bc.reshape(1, OUT)).T


def kernel(features, edge_index, W1, b1, W2, b2, Wc, bc):
    e = edge_index.shape[1]
    k_chunks = -(-e // (NW * C))
    k_chunks = -(-k_chunks // 8) * 8
    e_pad = NW * k_chunks * C
    assert e_pad % BLK == 0
    pck = _pack(edge_index.astype(jnp.int32), e_pad)
    t = _pre(features, W1, b1)
    parts, degs = _sc_aggregate(features, pck, k_chunks)
    return _dense(t, parts, degs, W1, W2, b2, Wc, bc)


# speedup vs baseline: 16.8701x; 1.0119x over previous
import functools

import jax
import jax.numpy as jnp
from jax import lax
from jax.experimental import pallas as pl
from jax.experimental.pallas import tpu as pltpu
from jax.experimental.pallas import tpu_sc as plsc

N_NODES = 10000
D_FEAT = 128
HIDDEN = 128
OUT = 64

NC = 2
NS = 16
NW = NC * NS
C = 128
ACC_ROWS = 10112
ROWS_PER_TILE = ACC_ROWS // NS
DEG_ROWS = 10240
DEG_PER_TILE = DEG_ROWS // NS
TRASH = N_NODES


def _sc_body(K, feat_hbm, pck_hbm, parts_hbm, degs_hbm,
             pck_v, src_i, dst_i, rows_v, ones_v, acc_sh, deg_sh,
             sem_g, sem_s, sem_d):
    c = lax.axis_index("c")
    s = lax.axis_index("s")
    wid = c * NS + s

    pltpu.async_copy(pck_hbm.at[pl.ds(wid * K * C, K * C)], pck_v, sem_g)

    zero16 = jnp.zeros((16,), jnp.float32)
    one16 = jnp.ones((16,), jnp.float32)
    for k in range(C // 16):
        ones_v[pl.ds(k * 16, 16)] = one16

    def _zero_row(i, carry):
        for k in range(C // 16):
            rows_v[0, i, pl.ds(k * 16, 16)] = zero16
        return carry

    lax.fori_loop(0, C, _zero_row, 0)

    zoffs = [min(o, ROWS_PER_TILE - C) for o in range(0, ROWS_PER_TILE, C)]
    for off in zoffs:
        pltpu.async_copy(rows_v.at[0],
                         acc_sh.at[pl.ds(s * ROWS_PER_TILE + off, C)], sem_s)
    for k in range(DEG_PER_TILE // C):
        pltpu.async_copy(rows_v.at[0, 0],
                         deg_sh.at[pl.ds(s * DEG_PER_TILE + k * C, C)], sem_d)
    for off in zoffs:
        pltpu.make_async_copy(
            rows_v.at[0],
            acc_sh.at[pl.ds(s * ROWS_PER_TILE + off, C)], sem_s).wait()
    for k in range(DEG_PER_TILE // C):
        pltpu.make_async_copy(
            rows_v.at[0, 0],
            deg_sh.at[pl.ds(s * DEG_PER_TILE + k * C, C)], sem_d).wait()
    pltpu.make_async_copy(pck_hbm.at[pl.ds(wid * K * C, K * C)], pck_v,
                          sem_g).wait()

    def _unpack(jj, b):
        for k in range(C // 16):
            w = pck_v[pl.ds(jj * C + k * 16, 16)]
            src_i[b, pl.ds(k * 16, 16)] = w & 0xFFFF
            dst_i[b, pl.ds(k * 16, 16)] = w >> 16

    _unpack(0, 0)
    pltpu.async_copy(feat_hbm.at[src_i.at[0]], rows_v.at[0], sem_g)
    plsc.subcore_barrier()

    def _wait_scatter(b):
        pltpu.make_async_copy(rows_v.at[b], acc_sh.at[dst_i.at[b]],
                              sem_s).wait()
        pltpu.make_async_copy(ones_v, deg_sh.at[dst_i.at[b]], sem_d).wait()

    def _chunk(j, carry):
        b = j & 1

        @pl.when(j >= 1)
        def _():
            _wait_scatter(1 - b)

        @pl.when(j + 1 < K)
        def _():
            _unpack(j + 1, 1 - b)
            pltpu.async_copy(feat_hbm.at[src_i.at[1 - b]],
                             rows_v.at[1 - b], sem_g)

        pltpu.make_async_copy(feat_hbm.at[src_i.at[b]], rows_v.at[b],
                              sem_g).wait()
        pltpu.async_copy(rows_v.at[b], acc_sh.at[dst_i.at[b]], sem_s,
                         add=True)
        pltpu.async_copy(ones_v, deg_sh.at[dst_i.at[b]], sem_d, add=True)
        return carry

    lax.fori_loop(0, K, _chunk, 0)
    _wait_scatter((K - 1) & 1)
    plsc.subcore_barrier()

    row0 = s * ROWS_PER_TILE
    pltpu.sync_copy(acc_sh.at[pl.ds(row0, ROWS_PER_TILE)],
                    parts_hbm.at[c, pl.ds(row0, ROWS_PER_TILE)])
    deg0 = s * DEG_PER_TILE
    pltpu.sync_copy(deg_sh.at[pl.ds(deg0, DEG_PER_TILE)],
                    degs_hbm.at[c, pl.ds(deg0, DEG_PER_TILE)])


def _sc_aggregate(features, pck2d, K):
    f = pl.kernel(
        functools.partial(_sc_body, K),
        out_type=[
            jax.ShapeDtypeStruct((NC, ACC_ROWS, D_FEAT), jnp.float32),
            jax.ShapeDtypeStruct((NC, DEG_ROWS), jnp.float32),
        ],
        mesh=plsc.VectorSubcoreMesh(core_axis_name="c", subcore_axis_name="s"),
        scratch_types=[
            pltpu.VMEM((K * C,), jnp.int32),
            pltpu.VMEM((2, C), jnp.int32),
            pltpu.VMEM((2, C), jnp.int32),
            pltpu.VMEM((2, C, D_FEAT), jnp.float32),
            pltpu.VMEM((C,), jnp.float32),
            pltpu.VMEM_SHARED((ACC_ROWS, D_FEAT), jnp.float32),
            pltpu.VMEM_SHARED((DEG_ROWS,), jnp.float32),
            pltpu.SemaphoreType.DMA,
            pltpu.SemaphoreType.DMA,
            pltpu.SemaphoreType.DMA,
        ],
    )
    return f(features, pck2d)


BLK = 40960


def _pack_body(e, ei_ref, o_ref):
    i = pl.program_id(0)
    real = (ei_ref[1] << 16) | ei_ref[0]

    @pl.when(i < e // BLK)
    def _():
        o_ref[...] = real

    @pl.when(i >= e // BLK)
    def _():
        gid = jax.lax.broadcasted_iota(jnp.int32, (BLK,), 0) + i * BLK
        fake = ((TRASH + (gid & 63)) << 16) | (gid & 8191)
        o_ref[...] = jnp.where(gid < e, real, fake)


def _pack(ei, e_pad):
    return pl.pallas_call(
        functools.partial(_pack_body, ei.shape[1]),
        grid=(e_pad // BLK,),
        in_specs=[pl.BlockSpec((2, BLK), lambda i: (0, i))],
        out_specs=pl.BlockSpec((BLK,), lambda i: (i,)),
        out_shape=jax.ShapeDtypeStruct((e_pad,), jnp.int32),
    )(ei)


BM = 2560


def _pre_body(f_ref, w1_ref, b1_ref, t_ref):
    t_ref[...] = jnp.dot(f_ref[...], w1_ref[:D_FEAT],
                         preferred_element_type=jnp.float32) + b1_ref[...]


def _pre(features, W1, b1):
    return pl.pallas_call(
        _pre_body,
        grid=(-(-N_NODES // BM),),
        in_specs=[
            pl.BlockSpec((BM, D_FEAT), lambda i: (i, 0)),
            pl.BlockSpec((2 * D_FEAT, HIDDEN), lambda i: (0, 0)),
            pl.BlockSpec((1, HIDDEN), lambda i: (0, 0)),
        ],
        out_specs=pl.BlockSpec((BM, HIDDEN), lambda i: (i, 0)),
        out_shape=jax.ShapeDtypeStruct((N_NODES, HIDDEN), jnp.float32),
    )(features, W1, b1.reshape(1, HIDDEN))


def _dense_body(t_ref, p_ref, d_ref, w1_ref, w2_ref, b2_ref,
                wc_ref, bc_ref, o_ref):
    agg = p_ref[0] + p_ref[1]
    deg = jnp.reshape(d_ref[0] + d_ref[1], (BM, 1))
    mean = agg / jnp.maximum(deg, 1.0)
    h = t_ref[...] + jnp.dot(mean, w1_ref[D_FEAT:],
                             preferred_element_type=jnp.float32)
    h = jnp.maximum(h, 0.0)
    h = jnp.dot(h, w2_ref[...], preferred_element_type=jnp.float32)
    h = jnp.maximum(h + b2_ref[...], 0.0)
    res = jnp.dot(h, wc_ref[...],
                  preferred_element_type=jnp.float32) + bc_ref[...]
    o_ref[...] = res.T


def _dense(t, parts, degs, W1, W2, b2, Wc, bc):
    return pl.pallas_call(
        _dense_body,
        grid=(-(-N_NODES // BM),),
        in_specs=[
            pl.BlockSpec((BM, HIDDEN), lambda i: (i, 0)),
            pl.BlockSpec((NC, BM, D_FEAT), lambda i: (0, i, 0)),
            pl.BlockSpec((NC, BM), lambda i: (0, i)),
            pl.BlockSpec((2 * D_FEAT, HIDDEN), lambda i: (0, 0)),
            pl.BlockSpec((HIDDEN, HIDDEN), lambda i: (0, 0)),
            pl.BlockSpec((1, HIDDEN), lambda i: (0, 0)),
            pl.BlockSpec((HIDDEN, OUT), lambda i: (0, 0)),
            pl.BlockSpec((1, OUT), lambda i: (0, 0)),
        ],
        out_specs=pl.BlockSpec((OUT, BM), lambda i: (0, i)),
        out_shape=jax.ShapeDtypeStruct((OUT, N_NODES), jnp.float32),
    )(t, parts, degs, W1, W2, b2.reshape(1, HIDDEN),
      Wc, bc.reshape(1, OUT)).T


def kernel(features, edge_index, W1, b1, W2, b2, Wc, bc):
    e = edge_index.shape[1]
    k_chunks = -(-e // (NW * C))
    k_chunks = -(-k_chunks // 8) * 8
    e_pad = NW * k_chunks * C
    assert e_pad % BLK == 0
    pck = _pack(edge_index.astype(jnp.int32), e_pad)
    t = _pre(features, W1, b1)
    parts, degs = _sc_aggregate(features, pck, k_chunks)
    return _dense(t, parts, degs, W1, W2, b2, Wc, bc)
